# baseline (speedup 1.0000x reference)
import functools

import jax
import jax.numpy as jnp
from jax import lax
from jax.experimental import pallas as pl
from jax.experimental.pallas import tpu as pltpu
from jax.experimental.pallas import tpu_sc as plsc

B = 1024
T = 200
D = 128
V = 1000
DP = D // 2

NC = 2
NS = 16
NW = NC * NS
BPW = B // NW

LANES = 16
NG = (T + LANES - 1) // LANES
NFULL = NG - 1
TAIL = T - NFULL * LANES
UNROLL = 16


def _make_sc_kernel():
    mesh = plsc.VectorSubcoreMesh(core_axis_name="c", subcore_axis_name="s")

    @functools.partial(
        pl.kernel,
        mesh=mesh,
        out_type=jax.ShapeDtypeStruct((B, D, T), jnp.float32),
        compiler_params=pltpu.CompilerParams(needs_layout_passes=False),
        scratch_types=[
            pltpu.VMEM((DP * V,), jnp.int32),
            pltpu.VMEM((BPW * T + LANES,), jnp.int32),
            pltpu.VMEM((D // 2, T), jnp.float32),
            pltpu.VMEM((D // 2, T), jnp.float32),
            pltpu.SemaphoreType.DMA,
            pltpu.SemaphoreType.DMA,
        ],
    )
    def k(idx_hbm, tabp_hbm, out_hbm, tab_v, idx_v, tile0, tile1, sem0, sem1):
        wid = lax.axis_index("s") * NC + lax.axis_index("c")
        lane = lax.iota(jnp.int32, 16)
        tail_mask = lane < TAIL
        sh16 = jnp.full((16,), 16, jnp.int32)
        himask = jnp.full((16,), -65536, jnp.int32)

        idx_v[pl.ds(BPW * T, LANES)] = jnp.zeros((16,), jnp.int32)
        pltpu.sync_copy(
            idx_hbm.at[pl.ds(wid * BPW * T, BPW * T)],
            idx_v.at[pl.ds(0, BPW * T)],
        )
        pltpu.sync_copy(tabp_hbm, tab_v)

        def unpack(x):
            even = plsc.bitcast(lax.shift_left(x, sh16), jnp.float32)
            odd = plsc.bitcast(lax.bitwise_and(x, himask), jnp.float32)
            return even, odd

        def drain(tile, sem):
            pltpu.make_async_copy(
                tile, out_hbm.at[0, pl.ds(0, D // 2), :], sem
            ).wait()

        DP2 = DP // 2

        def gather_tile(i, tile, c):
            def body_g(g, _, c=c):
                idxv = idx_v[pl.ds(i * T + g * LANES, LANES)]

                def body_d(blk, _, c=c):
                    lp0 = blk * UNROLL
                    xs = [
                        plsc.load_gather(
                            tab_v.at[pl.ds((c * DP2 + lp0 + u) * V, V)], [idxv]
                        )
                        for u in range(UNROLL)
                    ]
                    for u in range(UNROLL):
                        even, odd = unpack(xs[u])
                        tile[2 * (lp0 + u), pl.ds(g * LANES, LANES)] = even
                        tile[2 * (lp0 + u) + 1, pl.ds(g * LANES, LANES)] = odd
                    return 0

                lax.fori_loop(0, DP2 // UNROLL, body_d, 0)
                return 0

            lax.fori_loop(0, NFULL, body_g, 0)

            idxv = idx_v[pl.ds(i * T + NFULL * LANES, LANES)]
            tvec = NFULL * LANES + lane

            def body_dt(blk, _, c=c):
                lp0 = blk * UNROLL
                xs = [
                    plsc.load_gather(
                        tab_v.at[pl.ds((c * DP2 + lp0 + u) * V, V)], [idxv],
                        mask=tail_mask,
                    )
                    for u in range(UNROLL)
                ]
                for u in range(UNROLL):
                    even, odd = unpack(xs[u])
                    dv0 = jnp.full((16,), 2 * (lp0 + u), jnp.int32)
                    dv1 = jnp.full((16,), 2 * (lp0 + u) + 1, jnp.int32)
                    plsc.store_scatter(tile, [dv0, tvec], even, mask=tail_mask)
                    plsc.store_scatter(tile, [dv1, tvec], odd, mask=tail_mask)
                return 0

            lax.fori_loop(0, DP2 // UNROLL, body_dt, 0)

        for c in range(2):
            def body_v(vv, _, c=c):
                for p, tile, sem in ((0, tile0, sem0), (1, tile1, sem1)):
                    if c == 0:
                        @pl.when(vv > 0)
                        def _():
                            drain(tile, sem)
                    else:
                        drain(tile, sem)
                    i = 2 * vv + p
                    gather_tile(i, tile, c)
                    b = wid * BPW + i
                    pltpu.async_copy(
                        tile, out_hbm.at[b, pl.ds(c * (D // 2), D // 2), :], sem
                    )
                return 0

            lax.fori_loop(0, BPW // 2, body_v, 0)

        drain(tile0, sem0)
        drain(tile1, sem1)

    return k


_sc_kernel = _make_sc_kernel()


def _transpose_body(x_ref, o_ref):
    o_ref[...] = jnp.transpose(x_ref[...], (1, 0))


_tc_transpose = pl.pallas_call(
    _transpose_body,
    out_shape=jax.ShapeDtypeStruct((DP, V), jnp.int32),
)


def kernel(inputs, embedding_table):
    pairs = jax.lax.bitcast_convert_type(
        embedding_table.astype(jnp.bfloat16).reshape(V, DP, 2), jnp.int32
    )
    tabp = _tc_transpose(pairs).reshape(DP * V)
    return _sc_kernel(inputs.reshape(B * T), tabp)


# speedup vs baseline: 1.0099x; 1.0099x over previous
import functools

import jax
import jax.numpy as jnp
from jax import lax
from jax.experimental import pallas as pl
from jax.experimental.pallas import tpu as pltpu
from jax.experimental.pallas import tpu_sc as plsc

B = 1024
T = 200
D = 128
V = 1000
DP = D // 2

NC = 2
NS = 16
NW = NC * NS
BPW = B // NW

LANES = 16
NG = (T + LANES - 1) // LANES
NFULL = NG - 1
TAIL = T - NFULL * LANES
UNROLL = 8


def _make_sc_kernel():
    mesh = plsc.VectorSubcoreMesh(core_axis_name="c", subcore_axis_name="s")

    @functools.partial(
        pl.kernel,
        mesh=mesh,
        out_type=jax.ShapeDtypeStruct((B, D, T), jnp.float32),
        compiler_params=pltpu.CompilerParams(needs_layout_passes=False),
        scratch_types=[
            pltpu.VMEM((DP * V,), jnp.int32),
            pltpu.VMEM((BPW * T + LANES,), jnp.int32),
            pltpu.VMEM((D // 2, T), jnp.float32),
            pltpu.VMEM((D // 2, T), jnp.float32),
            pltpu.SemaphoreType.DMA,
            pltpu.SemaphoreType.DMA,
        ],
    )
    def k(idx_hbm, tabp_hbm, out_hbm, tab_v, idx_v, tile0, tile1, sem0, sem1):
        wid = lax.axis_index("s") * NC + lax.axis_index("c")
        lane = lax.iota(jnp.int32, 16)
        tail_mask = lane < TAIL
        sh16 = jnp.full((16,), 16, jnp.int32)
        himask = jnp.full((16,), -65536, jnp.int32)

        idx_v[pl.ds(BPW * T, LANES)] = jnp.zeros((16,), jnp.int32)
        pltpu.sync_copy(
            idx_hbm.at[pl.ds(wid * BPW * T, BPW * T)],
            idx_v.at[pl.ds(0, BPW * T)],
        )
        pltpu.sync_copy(tabp_hbm, tab_v)

        def unpack(x):
            even = plsc.bitcast(lax.shift_left(x, sh16), jnp.float32)
            odd = plsc.bitcast(lax.bitwise_and(x, himask), jnp.float32)
            return even, odd

        def drain(tile, sem):
            pltpu.make_async_copy(
                tile, out_hbm.at[0, pl.ds(0, D // 2), :], sem
            ).wait()

        DP2 = DP // 2

        def gather_tile(i, tile, c):
            def body_g(g, _, c=c):
                idxv = idx_v[pl.ds(i * T + g * LANES, LANES)]

                def body_d(blk, _, c=c):
                    lp0 = blk * UNROLL
                    xs = [
                        plsc.load_gather(
                            tab_v.at[pl.ds((c * DP2 + lp0 + u) * V, V)], [idxv]
                        )
                        for u in range(UNROLL)
                    ]
                    for u in range(UNROLL):
                        even, odd = unpack(xs[u])
                        tile[2 * (lp0 + u), pl.ds(g * LANES, LANES)] = even
                        tile[2 * (lp0 + u) + 1, pl.ds(g * LANES, LANES)] = odd
                    return 0

                lax.fori_loop(0, DP2 // UNROLL, body_d, 0)
                return 0

            lax.fori_loop(0, NFULL, body_g, 0)

            idxv = idx_v[pl.ds(i * T + NFULL * LANES, LANES)]
            tvec = NFULL * LANES + lane

            def body_dt(blk, _, c=c):
                lp0 = blk * UNROLL
                xs = [
                    plsc.load_gather(
                        tab_v.at[pl.ds((c * DP2 + lp0 + u) * V, V)], [idxv],
                        mask=tail_mask,
                    )
                    for u in range(UNROLL)
                ]
                for u in range(UNROLL):
                    even, odd = unpack(xs[u])
                    dv0 = jnp.full((16,), 2 * (lp0 + u), jnp.int32)
                    dv1 = jnp.full((16,), 2 * (lp0 + u) + 1, jnp.int32)
                    plsc.store_scatter(tile, [dv0, tvec], even, mask=tail_mask)
                    plsc.store_scatter(tile, [dv1, tvec], odd, mask=tail_mask)
                return 0

            lax.fori_loop(0, DP2 // UNROLL, body_dt, 0)

        for c in range(2):
            def body_v(vv, _, c=c):
                for p, tile, sem in ((0, tile0, sem0), (1, tile1, sem1)):
                    if c == 0:
                        @pl.when(vv > 0)
                        def _():
                            drain(tile, sem)
                    else:
                        drain(tile, sem)
                    i = 2 * vv + p
                    gather_tile(i, tile, c)
                    b = wid * BPW + i
                    pltpu.async_copy(
                        tile, out_hbm.at[b, pl.ds(c * (D // 2), D // 2), :], sem
                    )
                return 0

            lax.fori_loop(0, BPW // 2, body_v, 0)

        drain(tile0, sem0)
        drain(tile1, sem1)

    return k


_sc_kernel = _make_sc_kernel()


def _transpose_body(x_ref, o_ref):
    o_ref[...] = jnp.transpose(x_ref[...], (1, 0))


_tc_transpose = pl.pallas_call(
    _transpose_body,
    out_shape=jax.ShapeDtypeStruct((DP, V), jnp.int32),
)


def kernel(inputs, embedding_table):
    pairs = jax.lax.bitcast_convert_type(
        embedding_table.astype(jnp.bfloat16).reshape(V, DP, 2), jnp.int32
    )
    tabp = _tc_transpose(pairs).reshape(DP * V)
    return _sc_kernel(inputs.reshape(B * T), tabp)
